# Initial kernel scaffold; baseline (speedup 1.0000x reference)
import jax
import jax.numpy as jnp
from jax.experimental import pallas as pl


def kernel(x, edge_index, edge_attr, beta):
    raise NotImplementedError("write your pallas kernel here")


# speedup vs baseline: 5.8115x; 5.8115x over previous
import jax
import jax.numpy as jnp
from jax import lax
from jax.experimental import pallas as pl
from jax.experimental.pallas import tpu as pltpu
from jax.experimental.pallas import tpu_sc as plsc

N = 10000
K = 4
E = 320000
ITERS = 10
NS = 16
NPT = 640
N2 = NS * NPT
CH = 2048
LANES = 16

_LOG_C = (0.9999942730825785, -0.4998385694205269, 0.33154865896808255,
          -0.2398262845376411, 0.16582295421804527, -0.09325222046757188,
          0.03484979586898768, -0.006151485803420071)
_LN2 = 0.6931471805599453


def _log_onep(y):
    bits = lax.bitcast_convert_type(y, jnp.int32)
    e = (bits >> 23) - 127
    m = lax.bitcast_convert_type(
        jnp.bitwise_or(jnp.bitwise_and(bits, 0x007FFFFF), 0x3F800000),
        jnp.float32)
    r = m - 1.0
    p = jnp.float32(_LOG_C[7])
    for c in _LOG_C[6::-1]:
        p = p * r + jnp.float32(c)
    p = p * r
    return e.astype(jnp.float32) * jnp.float32(_LN2) + p


def _take(v, idx):
    return jnp.take_along_axis(v, idx, axis=0)


def _body(src_h, dst_h, w_h, b0_h, est_h, een_h, beta_h,
          sp_out, q_out, spart, sred,
          b_loc, src_c, dst_c, w_c, agg, deg,
          est_v, een_v, beta_vr, tmp16, part):
    wid = lax.axis_index("s")
    iota = lax.iota(jnp.int32, LANES)
    zeros16 = jnp.zeros((LANES,), jnp.float32)

    pltpu.sync_copy(est_h, est_v)
    pltpu.sync_copy(een_h, een_v)
    pltpu.sync_copy(beta_h, beta_vr)
    pltpu.sync_copy(b0_h, b_loc)
    beta = beta_vr[...]
    lo_e = est_v[pl.ds(wid, LANES)][0]
    hi_e = een_v[pl.ds(wid, LANES)][0]
    lo_n = wid * NPT
    c_lo = lo_e >> 11
    c_hi = (hi_e + (CH - 1)) >> 11

    def edge_sweep(need_src, win_fn, carry0):
        def chunk_body(c, carry):
            cbase = c * CH
            if need_src:
                pltpu.sync_copy(src_h.at[pl.ds(cbase, CH)], src_c)
            pltpu.sync_copy(dst_h.at[pl.ds(cbase, CH)], dst_c)
            pltpu.sync_copy(w_h.at[pl.ds(cbase, CH)], w_c)
            wlo = (jnp.maximum(lo_e, cbase) - cbase) >> 4
            whi = (jnp.minimum(hi_e, cbase + CH) - cbase + (LANES - 1)) >> 4

            def win_body(wi, carry):
                base = wi * LANES
                dvec = dst_c[pl.ds(base, LANES)]
                wvec = w_c[pl.ds(base, LANES)]
                eid = cbase + base + iota
                valid = (eid >= lo_e) & (eid < hi_e)
                return win_fn(base, dvec, wvec, valid, carry)

            return lax.fori_loop(wlo, whi, win_body, carry)

        return lax.fori_loop(c_lo, c_hi, chunk_body, carry0)

    def seg_parts(dvec):
        dprev = _take(dvec, jnp.maximum(iota - 1, 0))
        bm = (iota == 0) | (dvec != dprev)
        lastb_f = plsc.cummax(jnp.where(bm, iota.astype(jnp.float32), 0.0))
        dnext = _take(dvec, jnp.minimum(iota + 1, LANES - 1))
        endm = (iota == LANES - 1) | (dvec != dnext)
        lastb = lastb_f.astype(jnp.int32)
        previ = jnp.maximum(lastb - 1, 0)
        has_prev = lastb > 0
        return previ, has_prev, endm

    def seg_sum(vals, previ, has_prev):
        cs = plsc.cumsum(vals)
        cprev = _take(cs, previ)
        return cs - jnp.where(has_prev, cprev, 0.0)

    def zero_deg(j, _):
        deg[pl.ds(j * LANES, LANES)] = zeros16
        return 0

    lax.fori_loop(0, NPT // LANES, zero_deg, jnp.int32(0))

    def deg_win(base, dvec, wvec, valid, acc):
        val = jnp.where(valid, wvec, 0.0)
        previ, has_prev, endm = seg_parts(dvec)
        seg = seg_sum(val, previ, has_prev)
        wm = endm & valid
        drel = jnp.where(wm, dvec - lo_n, 0)
        plsc.addupdate_scatter(deg, [drel], seg, mask=wm)
        return acc + val

    acc2m = edge_sweep(False, deg_win, zeros16)
    tmp16[...] = acc2m
    pltpu.sync_copy(tmp16, sred.at[wid])
    plsc.subcore_barrier()
    pltpu.sync_copy(sred, part)
    tot = zeros16
    for j in range(NS):
        tot = tot + part[j]
    two_m = jnp.full((LANES,), jnp.sum(tot), jnp.float32)
    inv2m = 1.0 / two_m
    nbeta_inv = -beta * inv2m

    def iter_body(it, _):
        def pf_body(j, pf):
            dv = deg[pl.ds(j * LANES, LANES)]
            return tuple(
                pf[k] + dv * b_loc[pl.ds(k * N2 + lo_n + j * LANES, LANES)]
                for k in range(K))

        pf = lax.fori_loop(0, NPT // LANES, pf_body, (zeros16,) * K)
        fv = zeros16
        for k in range(K):
            fv = jnp.where(iota == k,
                           jnp.full((LANES,), jnp.sum(pf[k]), jnp.float32), fv)
        tmp16[...] = fv
        pltpu.sync_copy(tmp16, spart.at[wid])

        def zero_agg(j, _):
            for k in range(K):
                agg[pl.ds(k * NPT + j * LANES, LANES)] = zeros16
            return 0

        lax.fori_loop(0, NPT // LANES, zero_agg, jnp.int32(0))

        def msg_win(base, dvec, wvec, valid, carry):
            srcv = src_c[pl.ds(base, LANES)]
            ew = jnp.exp(beta * wvec) - 1.0
            previ, has_prev, endm = seg_parts(dvec)
            wm = endm & valid
            drel = jnp.where(wm, dvec - lo_n, 0)
            for k in range(K):
                bs = plsc.load_gather(b_loc, [srcv + (k * N2)])
                msg = _log_onep(1.0 + ew * bs)
                msg = jnp.where(valid, msg, 0.0)
                seg = seg_sum(msg, previ, has_prev)
                plsc.addupdate_scatter(agg, [drel + (k * NPT)], seg, mask=wm)
            return carry

        edge_sweep(True, msg_win, jnp.int32(0))
        plsc.subcore_barrier()

        pltpu.sync_copy(spart, part)
        ptot = zeros16
        for j in range(NS):
            ptot = ptot + part[j]
        fieldk = []
        for k in range(K):
            fk = jnp.full((LANES,), jnp.sum(jnp.where(iota == k, ptot, 0.0)),
                          jnp.float32)
            fieldk.append(nbeta_inv * fk)

        def node_body(j, _):
            dv = deg[pl.ds(j * LANES, LANES)]
            ls = [agg[pl.ds(k * NPT + j * LANES, LANES)] + dv * fieldk[k]
                  for k in range(K)]
            mx = jnp.maximum(jnp.maximum(ls[0], ls[1]),
                             jnp.maximum(ls[2], ls[3]))
            es = [jnp.exp(l - mx) for l in ls]
            inv = 1.0 / ((es[0] + es[1]) + (es[2] + es[3]))
            for k in range(K):
                bo = b_loc[pl.ds(k * N2 + lo_n + j * LANES, LANES)]
                agg[pl.ds(k * NPT + j * LANES, LANES)] = (
                    0.5 * bo + 0.5 * es[k] * inv)
            return 0

        lax.fori_loop(0, NPT // LANES, node_body, jnp.int32(0))

        for k in range(K):
            pltpu.sync_copy(agg.at[pl.ds(k * NPT, NPT)],
                            sp_out.at[pl.ds(k * N2 + lo_n, NPT)])
        plsc.subcore_barrier()
        pltpu.sync_copy(sp_out, b_loc)
        return 0

    lax.fori_loop(0, ITERS, iter_body, jnp.int32(0))

    def q_win(base, dvec, wvec, valid, acc):
        srcv = src_c[pl.ds(base, LANES)]
        dot = zeros16
        for k in range(K):
            ss = plsc.load_gather(b_loc, [srcv + (k * N2)])
            sd = plsc.load_gather(b_loc, [dvec + (k * N2)])
            dot = dot + ss * sd
        return acc + jnp.where(valid, wvec * dot, 0.0)

    qacc = edge_sweep(True, q_win, zeros16)

    def cd_body(j, pf):
        dv = deg[pl.ds(j * LANES, LANES)]
        return tuple(
            pf[k] + dv * b_loc[pl.ds(k * N2 + lo_n + j * LANES, LANES)]
            for k in range(K))

    cds = lax.fori_loop(0, NPT // LANES, cd_body, (zeros16,) * K)
    row = jnp.where(iota == 0,
                    jnp.full((LANES,), jnp.sum(qacc), jnp.float32), zeros16)
    for k in range(K):
        row = jnp.where(iota == k + 1,
                        jnp.full((LANES,), jnp.sum(cds[k]), jnp.float32), row)
    tmp16[...] = row
    pltpu.sync_copy(tmp16, sred.at[wid])
    plsc.subcore_barrier()

    @pl.when(wid == 0)
    def _():
        pltpu.sync_copy(sred, part)
        t2 = zeros16
        for j in range(NS):
            t2 = t2 + part[j]
        qe = jnp.full((LANES,), jnp.sum(jnp.where(iota == 0, t2, 0.0)),
                      jnp.float32)
        qv = qe * inv2m
        for k in range(K):
            cdk = jnp.full((LANES,),
                           jnp.sum(jnp.where(iota == k + 1, t2, 0.0)),
                           jnp.float32) * inv2m
            qv = qv - cdk * cdk
        tmp16[...] = qv
        pltpu.sync_copy(tmp16, q_out)


def kernel(x, edge_index, edge_attr, beta):
    del x
    src = edge_index[0]
    dst = edge_index[1]
    w = edge_attr[:, 0].astype(jnp.float32)
    order = jnp.argsort(dst)
    srcs = src[order].astype(jnp.int32)
    dsts = dst[order].astype(jnp.int32)
    ws = w[order]
    ep = ((E + CH - 1) // CH) * CH
    pad = ep - E
    srcs = jnp.concatenate([srcs, jnp.zeros((pad,), jnp.int32)])
    dsts = jnp.concatenate([dsts, jnp.full((pad,), N2 - 1, jnp.int32)])
    ws = jnp.concatenate([ws, jnp.zeros((pad,), jnp.float32)])
    bnd = jnp.searchsorted(
        dsts, (jnp.arange(17, dtype=jnp.int32) * NPT).astype(jnp.int32)
    ).astype(jnp.int32)
    est = jnp.zeros((2 * LANES,), jnp.int32).at[:16].set(bnd[:16])
    een = jnp.zeros((2 * LANES,), jnp.int32).at[:16].set(bnd[1:])

    init_logits = 0.1 * jnp.sin(
        jnp.arange(N * K, dtype=jnp.float32).reshape(N, K) * 0.37)
    b0 = jax.nn.softmax(init_logits, axis=-1)
    b0p = jnp.full((K, N2), 0.25, jnp.float32).at[:, :N].set(b0.T).reshape(-1)
    beta_v = jnp.full((LANES,), beta, jnp.float32)

    mesh = plsc.VectorSubcoreMesh(
        core_axis_name="c", subcore_axis_name="s", num_cores=1)
    f = pl.kernel(
        _body,
        out_type=[jax.ShapeDtypeStruct((K * N2,), jnp.float32),
                  jax.ShapeDtypeStruct((LANES,), jnp.float32),
                  jax.ShapeDtypeStruct((NS, LANES), jnp.float32),
                  jax.ShapeDtypeStruct((NS, LANES), jnp.float32)],
        mesh=mesh,
        compiler_params=pltpu.CompilerParams(needs_layout_passes=False),
        scratch_types=[
            pltpu.VMEM((K * N2,), jnp.float32),
            pltpu.VMEM((CH,), jnp.int32),
            pltpu.VMEM((CH,), jnp.int32),
            pltpu.VMEM((CH,), jnp.float32),
            pltpu.VMEM((K * NPT,), jnp.float32),
            pltpu.VMEM((NPT,), jnp.float32),
            pltpu.VMEM((2 * LANES,), jnp.int32),
            pltpu.VMEM((2 * LANES,), jnp.int32),
            pltpu.VMEM((LANES,), jnp.float32),
            pltpu.VMEM((LANES,), jnp.float32),
            pltpu.VMEM((NS, LANES), jnp.float32),
        ],
    )
    sp, qv, _, _ = f(srcs, dsts, ws, b0p, est, een, beta_v)
    s = sp.reshape(K, N2)[:, :N].T
    return (s, qv[0])
